# Initial kernel scaffold; baseline (speedup 1.0000x reference)
import jax
import jax.numpy as jnp
from jax.experimental import pallas as pl


def kernel(x, edge_index, edge_attr, batch, W1, a1_src, a1_dst, b1, W2, a2_src, a2_dst, b2, W3, a3_src, a3_dst, b3, Wp, bp):
    raise NotImplementedError("write your pallas kernel here")


# speedup vs baseline: 1.5713x; 1.5713x over previous
import jax
import jax.numpy as jnp
from jax.experimental import pallas as pl

_BLK = 1000


def _elu(v):
    return jnp.where(v > 0, v, jnp.exp(jnp.minimum(v, 0.0)) - 1.0)


def _k1_body(x_ref, c_ref, al_ref):
    al_ref[...] = jnp.dot(x_ref[...], c_ref[...],
                          preferred_element_type=jnp.float32)


def _k2_body(s_ref, r_ref, w1_ref, b1_ref, w2_ref, a2_ref, h_ref, al_ref):
    sexp = jnp.dot(s_ref[...], r_ref[...], preferred_element_type=jnp.float32)
    g = _elu(sexp * w1_ref[...] + b1_ref[...])
    h = jnp.dot(g, w2_ref[...], preferred_element_type=jnp.float32)
    h_ref[...] = h
    al_ref[...] = jnp.dot(h, a2_ref[...], preferred_element_type=jnp.float32)


def _k3_body(hin_ref, b2_ref, w3_ref, a3_ref, h_ref, al_ref):
    g = _elu(hin_ref[...] + b2_ref[...])
    h = jnp.dot(g, w3_ref[...], preferred_element_type=jnp.float32)
    h_ref[...] = h
    al_ref[...] = jnp.dot(h, a3_ref[...], preferred_element_type=jnp.float32)


def _kc_body(h_ref, b3_ref, batch_ref, bvals_ref, wp_ref, bp_ref, out_ref):
    g = _elu(h_ref[...] + b3_ref[...])
    onehot = (bvals_ref[...] == batch_ref[...]).astype(jnp.float32)
    sums = jnp.dot(onehot, g, preferred_element_type=jnp.float32)
    cnt = jnp.sum(onehot, axis=1, keepdims=True)
    pooled = sums / jnp.maximum(cnt, 1.0)
    out_ref[...] = jnp.dot(pooled, wp_ref[...],
                           preferred_element_type=jnp.float32) + bp_ref[...]


def _row_spec(blk, d):
    return pl.BlockSpec((blk, d), lambda i: (i, 0))


def _full_spec(shape):
    return pl.BlockSpec(shape, lambda i: (0,) * len(shape))


def _whole_spec(shape):
    return pl.BlockSpec(shape, lambda: (0,) * len(shape))


def _softmax_coef(al_s, al_d, src, dst, n):
    alpha = al_s[src] + al_d[dst]
    alpha = jnp.where(alpha > 0, alpha, 0.2 * alpha)
    m = jax.ops.segment_max(alpha, dst, num_segments=n)
    m = jnp.where(jnp.isfinite(m), m, 0.0)
    ex = jnp.exp(alpha - m[dst])
    den = jax.ops.segment_sum(ex, dst, num_segments=n)
    return ex / (den[dst] + 1e-16)


def kernel(x, edge_index, edge_attr, batch, W1, a1_src, a1_dst, b1,
           W2, a2_src, a2_dst, b2, W3, a3_src, a3_dst, b3, Wp, bp):
    n, _ = x.shape
    heads = a1_src.shape[1]
    ch = a1_src.shape[2]
    hd = heads * ch

    sl = jnp.arange(n, dtype=edge_index.dtype)
    src = jnp.concatenate([edge_index[0], sl])
    dst = jnp.concatenate([edge_index[1], sl])

    eye_h = jnp.eye(heads, dtype=jnp.float32)

    w1h = W1.reshape(heads, ch)
    c_src = jnp.sum(w1h * a1_src[0], axis=-1)
    c_dst = jnp.sum(w1h * a1_dst[0], axis=-1)
    c_comb = jnp.concatenate([c_src, c_dst])[None, :]

    def mk_a(a):
        return (a[0][:, :, None] * eye_h[:, None, :]).reshape(hd, heads)

    a2c = jnp.concatenate([mk_a(a2_src), mk_a(a2_dst)], axis=1)
    a3c = jnp.concatenate(
        [a3_src[0, 0][:, None], a3_dst[0, 0][:, None],
         jnp.zeros((ch, 6), jnp.float32)], axis=1)
    r_exp = jnp.repeat(eye_h, ch, axis=1)

    grid = (n // _BLK,)

    al1 = pl.pallas_call(
        _k1_body,
        grid=grid,
        in_specs=[_row_spec(_BLK, 1), _full_spec((1, 2 * heads))],
        out_specs=_row_spec(_BLK, 2 * heads),
        out_shape=jax.ShapeDtypeStruct((n, 2 * heads), jnp.float32),
    )(x, c_comb)

    coef1 = _softmax_coef(al1[:, :heads], al1[:, heads:], src, dst, n)
    s1 = jax.ops.segment_sum(x[src] * coef1, dst, num_segments=n)

    h2, al2 = pl.pallas_call(
        _k2_body,
        grid=grid,
        in_specs=[_row_spec(_BLK, heads), _full_spec((heads, hd)),
                  _full_spec((1, hd)), _full_spec((1, hd)),
                  _full_spec((hd, hd)), _full_spec((hd, 2 * heads))],
        out_specs=[_row_spec(_BLK, hd), _row_spec(_BLK, 2 * heads)],
        out_shape=[jax.ShapeDtypeStruct((n, hd), jnp.float32),
                   jax.ShapeDtypeStruct((n, 2 * heads), jnp.float32)],
    )(s1, r_exp, W1.reshape(1, hd), b1.reshape(1, hd), W2, a2c)

    coef2 = _softmax_coef(al2[:, :heads], al2[:, heads:], src, dst, n)
    msg2 = h2[src].reshape(-1, heads, ch) * coef2[:, :, None]
    o2 = jax.ops.segment_sum(msg2, dst, num_segments=n).reshape(n, hd)

    h3, al3 = pl.pallas_call(
        _k3_body,
        grid=grid,
        in_specs=[_row_spec(_BLK, hd), _full_spec((1, hd)),
                  _full_spec((hd, ch)), _full_spec((ch, 8))],
        out_specs=[_row_spec(_BLK, ch), _row_spec(_BLK, 8)],
        out_shape=[jax.ShapeDtypeStruct((n, ch), jnp.float32),
                   jax.ShapeDtypeStruct((n, 8), jnp.float32)],
    )(o2, b2.reshape(1, hd), W3, a3c)

    coef3 = _softmax_coef(al3[:, :1], al3[:, 1:2], src, dst, n)
    o3 = jax.ops.segment_sum(h3[src] * coef3, dst, num_segments=n)

    bsz = 20
    npad = ((n + 127) // 128) * 128
    o3p = jnp.pad(o3, ((0, npad - n), (0, 0)))
    batch_row = jnp.pad(batch.astype(jnp.float32)[None, :],
                        ((0, 0), (0, npad - n)), constant_values=-1.0)
    bvals = jnp.arange(bsz, dtype=jnp.float32)[:, None]
    out = pl.pallas_call(
        _kc_body,
        in_specs=[_whole_spec((npad, ch)), _whole_spec((1, ch)),
                  _whole_spec((1, npad)), _whole_spec((bsz, 1)),
                  _whole_spec((ch, Wp.shape[1])), _whole_spec((1, Wp.shape[1]))],
        out_specs=_whole_spec((bsz, Wp.shape[1])),
        out_shape=jax.ShapeDtypeStruct((bsz, Wp.shape[1]), jnp.float32),
    )(o3p, b3.reshape(1, ch), batch_row, bvals, Wp, bp.reshape(1, -1))
    return out


# speedup vs baseline: 4.6814x; 2.9794x over previous
import jax
import jax.numpy as jnp
from jax.experimental import pallas as pl

_BLK = 1000


def _elu(v):
    return jnp.where(v > 0, v, jnp.exp(jnp.minimum(v, 0.0)) - 1.0)


def _k1_body(x_ref, c_ref, al_ref):
    al_ref[...] = jnp.dot(x_ref[...], c_ref[...],
                          preferred_element_type=jnp.float32)


def _k2_body(s_ref, r_ref, w1_ref, b1_ref, w2_ref, a2_ref, h_ref, al_ref):
    sexp = jnp.dot(s_ref[...], r_ref[...], preferred_element_type=jnp.float32)
    g = _elu(sexp * w1_ref[...] + b1_ref[...])
    h = jnp.dot(g, w2_ref[...], preferred_element_type=jnp.float32)
    h_ref[...] = h
    al_ref[...] = jnp.dot(h, a2_ref[...], preferred_element_type=jnp.float32)


def _k3_body(hin_ref, b2_ref, w3_ref, a3_ref, h_ref, al_ref):
    g = _elu(hin_ref[...] + b2_ref[...])
    h = jnp.dot(g, w3_ref[...], preferred_element_type=jnp.float32)
    h_ref[...] = h
    al_ref[...] = jnp.dot(h, a3_ref[...], preferred_element_type=jnp.float32)


def _kc_body(h_ref, b3_ref, batch_ref, bvals_ref, wp_ref, bp_ref, out_ref):
    g = _elu(h_ref[...] + b3_ref[...])
    onehot = (bvals_ref[...] == batch_ref[...]).astype(jnp.float32)
    sums = jnp.dot(onehot, g, preferred_element_type=jnp.float32)
    cnt = jnp.sum(onehot, axis=1, keepdims=True)
    pooled = sums / jnp.maximum(cnt, 1.0)
    out_ref[...] = jnp.dot(pooled, wp_ref[...],
                           preferred_element_type=jnp.float32) + bp_ref[...]


def _row_spec(blk, d):
    return pl.BlockSpec((blk, d), lambda i: (i, 0))


def _full_spec(shape):
    return pl.BlockSpec(shape, lambda i: (0,) * len(shape))


def _whole_spec(shape):
    return pl.BlockSpec(shape, lambda: (0,) * len(shape))


def _edge_exp(al_s, al_d, src, dst):
    mb = jnp.max(al_s, axis=0) + jnp.max(al_d, axis=0)
    mb = jnp.where(mb > 0, mb, 0.2 * mb)
    alpha = al_s[src] + al_d[dst]
    alpha = jnp.where(alpha > 0, alpha, 0.2 * alpha)
    return jnp.exp(alpha - mb[None, :])


def kernel(x, edge_index, edge_attr, batch, W1, a1_src, a1_dst, b1,
           W2, a2_src, a2_dst, b2, W3, a3_src, a3_dst, b3, Wp, bp):
    n, _ = x.shape
    heads = a1_src.shape[1]
    ch = a1_src.shape[2]
    hd = heads * ch

    sl = jnp.arange(n, dtype=edge_index.dtype)
    src = jnp.concatenate([edge_index[0], sl])
    dst = jnp.concatenate([edge_index[1], sl])

    eye_h = jnp.eye(heads, dtype=jnp.float32)

    w1h = W1.reshape(heads, ch)
    c_src = jnp.sum(w1h * a1_src[0], axis=-1)
    c_dst = jnp.sum(w1h * a1_dst[0], axis=-1)
    c_comb = jnp.concatenate([c_src, c_dst])[None, :]

    def mk_a(a):
        return (a[0][:, :, None] * eye_h[:, None, :]).reshape(hd, heads)

    a2c = jnp.concatenate([mk_a(a2_src), mk_a(a2_dst)], axis=1)
    a3c = jnp.concatenate(
        [a3_src[0, 0][:, None], a3_dst[0, 0][:, None],
         jnp.zeros((ch, 6), jnp.float32)], axis=1)
    r_exp = jnp.repeat(eye_h, ch, axis=1)

    grid = (n // _BLK,)

    al1 = pl.pallas_call(
        _k1_body,
        grid=grid,
        in_specs=[_row_spec(_BLK, 1), _full_spec((1, 2 * heads))],
        out_specs=_row_spec(_BLK, 2 * heads),
        out_shape=jax.ShapeDtypeStruct((n, 2 * heads), jnp.float32),
    )(x, c_comb)

    ex1 = _edge_exp(al1[:, :heads], al1[:, heads:], src, dst)
    agg1 = jax.ops.segment_sum(
        jnp.concatenate([x[src] * ex1, ex1], axis=1), dst, num_segments=n)
    s1 = agg1[:, :heads] / (agg1[:, heads:] + 1e-16)

    h2, al2 = pl.pallas_call(
        _k2_body,
        grid=grid,
        in_specs=[_row_spec(_BLK, heads), _full_spec((heads, hd)),
                  _full_spec((1, hd)), _full_spec((1, hd)),
                  _full_spec((hd, hd)), _full_spec((hd, 2 * heads))],
        out_specs=[_row_spec(_BLK, hd), _row_spec(_BLK, 2 * heads)],
        out_shape=[jax.ShapeDtypeStruct((n, hd), jnp.float32),
                   jax.ShapeDtypeStruct((n, 2 * heads), jnp.float32)],
    )(s1, r_exp, W1.reshape(1, hd), b1.reshape(1, hd), W2, a2c)

    ex2 = _edge_exp(al2[:, :heads], al2[:, heads:], src, dst)
    msg2 = (h2[src].reshape(-1, heads, ch) * ex2[:, :, None]).reshape(-1, hd)
    agg2 = jax.ops.segment_sum(
        jnp.concatenate([msg2, ex2], axis=1), dst, num_segments=n)
    o2 = (agg2[:, :hd].reshape(n, heads, ch)
          / (agg2[:, hd:, None] + 1e-16)).reshape(n, hd)

    h3, al3 = pl.pallas_call(
        _k3_body,
        grid=grid,
        in_specs=[_row_spec(_BLK, hd), _full_spec((1, hd)),
                  _full_spec((hd, ch)), _full_spec((ch, 8))],
        out_specs=[_row_spec(_BLK, ch), _row_spec(_BLK, 8)],
        out_shape=[jax.ShapeDtypeStruct((n, ch), jnp.float32),
                   jax.ShapeDtypeStruct((n, 8), jnp.float32)],
    )(o2, b2.reshape(1, hd), W3, a3c)

    ex3 = _edge_exp(al3[:, :1], al3[:, 1:2], src, dst)
    agg3 = jax.ops.segment_sum(
        jnp.concatenate([h3[src] * ex3, ex3], axis=1), dst, num_segments=n)
    o3 = agg3[:, :ch] / (agg3[:, ch:] + 1e-16)

    bsz = 20
    npad = ((n + 127) // 128) * 128
    o3p = jnp.pad(o3, ((0, npad - n), (0, 0)))
    batch_row = jnp.pad(batch.astype(jnp.float32)[None, :],
                        ((0, 0), (0, npad - n)), constant_values=-1.0)
    bvals = jnp.arange(bsz, dtype=jnp.float32)[:, None]
    out = pl.pallas_call(
        _kc_body,
        in_specs=[_whole_spec((npad, ch)), _whole_spec((1, ch)),
                  _whole_spec((1, npad)), _whole_spec((bsz, 1)),
                  _whole_spec((ch, Wp.shape[1])), _whole_spec((1, Wp.shape[1]))],
        out_specs=_whole_spec((bsz, Wp.shape[1])),
        out_shape=jax.ShapeDtypeStruct((bsz, Wp.shape[1]), jnp.float32),
    )(o3p, b3.reshape(1, ch), batch_row, bvals, Wp, bp.reshape(1, -1))
    return out


# speedup vs baseline: 5.9146x; 1.2634x over previous
import jax
import jax.numpy as jnp
from jax.experimental import pallas as pl

_BLK = 1000


def _elu(v):
    return jnp.where(v > 0, v, jnp.exp(jnp.minimum(v, 0.0)) - 1.0)


def _k1_body(x_ref, c_ref, al_ref):
    al_ref[...] = jnp.dot(x_ref[...], c_ref[...],
                          preferred_element_type=jnp.float32)


def _k2_body(s_ref, r_ref, w1_ref, b1_ref, w2_ref, a2_ref, h_ref, al_ref):
    sexp = jnp.dot(s_ref[...], r_ref[...], preferred_element_type=jnp.float32)
    g = _elu(sexp * w1_ref[...] + b1_ref[...])
    h = jnp.dot(g, w2_ref[...], preferred_element_type=jnp.float32)
    h_ref[...] = h
    al_ref[...] = jnp.dot(h, a2_ref[...], preferred_element_type=jnp.float32)


def _k3_body(hin_ref, b2_ref, w3_ref, a3_ref, h_ref, al_ref):
    g = _elu(hin_ref[...] + b2_ref[...])
    h = jnp.dot(g, w3_ref[...], preferred_element_type=jnp.float32)
    h_ref[...] = h
    al_ref[...] = jnp.dot(h, a3_ref[...], preferred_element_type=jnp.float32)


def _kc_body(h_ref, b3_ref, batch_ref, bvals_ref, wp_ref, bp_ref, out_ref):
    g = _elu(h_ref[...] + b3_ref[...])
    onehot = (bvals_ref[...] == batch_ref[...]).astype(jnp.float32)
    sums = jnp.dot(onehot, g, preferred_element_type=jnp.float32)
    cnt = jnp.sum(onehot, axis=1, keepdims=True)
    pooled = sums / jnp.maximum(cnt, 1.0)
    out_ref[...] = jnp.dot(pooled, wp_ref[...],
                           preferred_element_type=jnp.float32) + bp_ref[...]


def _row_spec(blk, d):
    return pl.BlockSpec((blk, d), lambda i: (i, 0))


def _full_spec(shape):
    return pl.BlockSpec(shape, lambda i: (0,) * len(shape))


def _whole_spec(shape):
    return pl.BlockSpec(shape, lambda: (0,) * len(shape))


def _edge_exp(als_e, ald_nodes, dst, max_s):
    mb = max_s + jnp.max(ald_nodes, axis=0)
    mb = jnp.where(mb > 0, mb, 0.2 * mb)
    alpha = als_e + ald_nodes[dst]
    alpha = jnp.where(alpha > 0, alpha, 0.2 * alpha)
    return jnp.exp(alpha - mb[None, :])


def kernel(x, edge_index, edge_attr, batch, W1, a1_src, a1_dst, b1,
           W2, a2_src, a2_dst, b2, W3, a3_src, a3_dst, b3, Wp, bp):
    n, _ = x.shape
    heads = a1_src.shape[1]
    ch = a1_src.shape[2]
    hd = heads * ch

    sl = jnp.arange(n, dtype=edge_index.dtype)
    src = jnp.concatenate([edge_index[0], sl])
    dst = jnp.concatenate([edge_index[1], sl])

    eye_h = jnp.eye(heads, dtype=jnp.float32)

    w1h = W1.reshape(heads, ch)
    c_src = jnp.sum(w1h * a1_src[0], axis=-1)
    c_dst = jnp.sum(w1h * a1_dst[0], axis=-1)
    c_comb = jnp.concatenate([c_src, c_dst])[None, :]

    def mk_a(a):
        return (a[0][:, :, None] * eye_h[:, None, :]).reshape(hd, heads)

    a2c = jnp.concatenate([mk_a(a2_src), mk_a(a2_dst)], axis=1)
    a3c = jnp.concatenate(
        [a3_src[0, 0][:, None], a3_dst[0, 0][:, None],
         jnp.zeros((ch, 6), jnp.float32)], axis=1)
    r_exp = jnp.repeat(eye_h, ch, axis=1)

    grid = (n // _BLK,)

    al1 = pl.pallas_call(
        _k1_body,
        grid=grid,
        in_specs=[_row_spec(_BLK, 1), _full_spec((1, 2 * heads))],
        out_specs=_row_spec(_BLK, 2 * heads),
        out_shape=jax.ShapeDtypeStruct((n, 2 * heads), jnp.float32),
    )(x, c_comb)

    xa1 = jnp.concatenate([x, al1[:, :heads]], axis=1)
    g1 = xa1[src]
    ex1 = _edge_exp(g1[:, 1:], al1[:, heads:], dst,
                    jnp.max(al1[:, :heads], axis=0))
    agg1 = jax.ops.segment_sum(
        jnp.concatenate([g1[:, :1] * ex1, ex1], axis=1), dst, num_segments=n)
    s1 = agg1[:, :heads] / (agg1[:, heads:] + 1e-16)

    h2, al2 = pl.pallas_call(
        _k2_body,
        grid=grid,
        in_specs=[_row_spec(_BLK, heads), _full_spec((heads, hd)),
                  _full_spec((1, hd)), _full_spec((1, hd)),
                  _full_spec((hd, hd)), _full_spec((hd, 2 * heads))],
        out_specs=[_row_spec(_BLK, hd), _row_spec(_BLK, 2 * heads)],
        out_shape=[jax.ShapeDtypeStruct((n, hd), jnp.float32),
                   jax.ShapeDtypeStruct((n, 2 * heads), jnp.float32)],
    )(s1, r_exp, W1.reshape(1, hd), b1.reshape(1, hd), W2, a2c)

    ha2 = jnp.concatenate([h2, al2[:, :heads]], axis=1)
    g2 = ha2[src]
    ex2 = _edge_exp(g2[:, hd:], al2[:, heads:], dst,
                    jnp.max(al2[:, :heads], axis=0))
    msg2 = (g2[:, :hd].reshape(-1, heads, ch) * ex2[:, :, None]).reshape(-1, hd)
    agg2 = jax.ops.segment_sum(
        jnp.concatenate([msg2, ex2], axis=1), dst, num_segments=n)
    o2 = (agg2[:, :hd].reshape(n, heads, ch)
          / (agg2[:, hd:, None] + 1e-16)).reshape(n, hd)

    h3, al3 = pl.pallas_call(
        _k3_body,
        grid=grid,
        in_specs=[_row_spec(_BLK, hd), _full_spec((1, hd)),
                  _full_spec((hd, ch)), _full_spec((ch, 8))],
        out_specs=[_row_spec(_BLK, ch), _row_spec(_BLK, 8)],
        out_shape=[jax.ShapeDtypeStruct((n, ch), jnp.float32),
                   jax.ShapeDtypeStruct((n, 8), jnp.float32)],
    )(o2, b2.reshape(1, hd), W3, a3c)

    ha3 = jnp.concatenate([h3, al3[:, :1]], axis=1)
    g3 = ha3[src]
    ex3 = _edge_exp(g3[:, ch:], al3[:, 1:2], dst,
                    jnp.max(al3[:, :1], axis=0))
    agg3 = jax.ops.segment_sum(
        jnp.concatenate([g3[:, :ch] * ex3, ex3], axis=1), dst, num_segments=n)
    o3 = agg3[:, :ch] / (agg3[:, ch:] + 1e-16)

    bsz = 20
    npad = ((n + 127) // 128) * 128
    o3p = jnp.pad(o3, ((0, npad - n), (0, 0)))
    batch_row = jnp.pad(batch.astype(jnp.float32)[None, :],
                        ((0, 0), (0, npad - n)), constant_values=-1.0)
    bvals = jnp.arange(bsz, dtype=jnp.float32)[:, None]
    out = pl.pallas_call(
        _kc_body,
        in_specs=[_whole_spec((npad, ch)), _whole_spec((1, ch)),
                  _whole_spec((1, npad)), _whole_spec((bsz, 1)),
                  _whole_spec((ch, Wp.shape[1])), _whole_spec((1, Wp.shape[1]))],
        out_specs=_whole_spec((bsz, Wp.shape[1])),
        out_shape=jax.ShapeDtypeStruct((bsz, Wp.shape[1]), jnp.float32),
    )(o3p, b3.reshape(1, ch), batch_row, bvals, Wp, bp.reshape(1, -1))
    return out
